# baseline (speedup 1.0000x reference)
import jax
import jax.numpy as jnp
from jax.experimental import pallas as pl
from jax.experimental.pallas import tpu as pltpu

_N = 2048
_NUM_PHYS = 600000
_THRESH = 1.0
_BLK = 256
_NSTRIP = _N // _BLK


def _notch_kernel(xc_ref, yc_ref, hxc_ref, hyc_ref,
                  xr_ref, yr_ref, hxr_ref, hyr_ref, out_ref):
    r = pl.program_id(0)
    base = r * _BLK
    xc = xc_ref[...]
    yc = yc_ref[...]
    hxc = hxc_ref[...]
    hyc = hyc_ref[...]

    def block(cb):
        xr = xr_ref[:, pl.ds(cb, _BLK)]
        yr = yr_ref[:, pl.ds(cb, _BLK)]
        hxr = hxr_ref[:, pl.ds(cb, _BLK)]
        hyr = hyr_ref[:, pl.ds(cb, _BLK)]
        dx = jnp.maximum(jnp.abs(xc - xr) - (hxc + hxr), 0.0)
        dy = jnp.maximum(jnp.abs(yc - yr) - (hyc + hyr), 0.0)
        p = jnp.maximum((_THRESH - dx) - dy, 0.0)
        return p * p

    lrow = jax.lax.broadcasted_iota(jnp.int32, (_BLK, _BLK), 0)
    lcol = jax.lax.broadcasted_iota(jnp.int32, (_BLK, _BLK), 1)
    acc = jnp.sum(jnp.where(lcol > lrow, block(base), 0.0))

    def body(c, a):
        return a + jnp.sum(block(c * _BLK))

    acc = jax.lax.fori_loop(r + 1, _NSTRIP, body, acc)
    out_ref[0, 0, 0] = acc


def kernel(pos, macro_mask, macro_size_x, macro_size_y):
    x = jax.lax.slice(pos, (0,), (_N,))
    y = jax.lax.slice(pos, (_NUM_PHYS,), (_NUM_PHYS + _N,))
    m = macro_mask
    neg = jnp.where(m, jnp.float32(0.0), jnp.float32(-1e7))
    hx = macro_size_x.astype(jnp.float32) * 0.5 + neg
    hy = macro_size_y.astype(jnp.float32) * 0.5 + neg

    col = lambda v: v.reshape(_N, 1)
    row = lambda v: v.reshape(1, _N)

    partial = pl.pallas_call(
        _notch_kernel,
        grid=(_NSTRIP,),
        in_specs=[
            pl.BlockSpec((_BLK, 1), lambda r: (r, 0)),
            pl.BlockSpec((_BLK, 1), lambda r: (r, 0)),
            pl.BlockSpec((_BLK, 1), lambda r: (r, 0)),
            pl.BlockSpec((_BLK, 1), lambda r: (r, 0)),
            pl.BlockSpec((1, _N), lambda r: (0, 0)),
            pl.BlockSpec((1, _N), lambda r: (0, 0)),
            pl.BlockSpec((1, _N), lambda r: (0, 0)),
            pl.BlockSpec((1, _N), lambda r: (0, 0)),
        ],
        out_shape=jax.ShapeDtypeStruct((_NSTRIP, 1, 1), jnp.float32),
        out_specs=pl.BlockSpec((1, 1, 1), lambda r: (r, 0, 0),
                               memory_space=pltpu.SMEM),
        compiler_params=pltpu.CompilerParams(
            dimension_semantics=("parallel",)),
    )(col(x), col(y), col(hx), col(hy), row(x), row(y), row(hx), row(hy))

    total = jnp.sum(partial)
    count = jnp.sum(m.astype(jnp.int32))
    return jnp.where(count < 2, jnp.zeros((), jnp.float32), total)


# speedup vs baseline: 1.1631x; 1.1631x over previous
import jax
import jax.numpy as jnp
from jax.experimental import pallas as pl
from jax.experimental.pallas import tpu as pltpu

_N = 2048
_NUM_PHYS = 600000
_THRESH = 1.0
_BLK = 256
_HALF = _N // 2
_MID = _HALF - _BLK
_NSTRIP = _N // _BLK


def _notch_kernel(xc_ref, yc_ref, hxc_ref, hyc_ref,
                  xr_ref, yr_ref, hxr_ref, hyr_ref, out_ref):
    r = pl.program_id(0)
    base = r * _BLK
    xc = xc_ref[...]
    yc = yc_ref[...]
    hxc = hxc_ref[...]
    hyc = hyc_ref[...]

    def p2(co, w):
        xr = xr_ref[:, pl.ds(co, w)]
        yr = yr_ref[:, pl.ds(co, w)]
        hxr = hxr_ref[:, pl.ds(co, w)]
        hyr = hyr_ref[:, pl.ds(co, w)]
        dx = jnp.maximum(jnp.abs(xc - xr) - (hxc + hxr), 0.0)
        dy = jnp.maximum(jnp.abs(yc - yr) - (hyc + hyr), 0.0)
        p = jnp.maximum((_THRESH - dx) - dy, 0.0)
        return p * p

    lrow = jax.lax.broadcasted_iota(jnp.int32, (_BLK, _BLK), 0)
    lcol = jax.lax.broadcasted_iota(jnp.int32, (_BLK, _BLK), 1)

    s = jnp.sum(jnp.where(lcol > lrow, p2(base, _BLK), 0.0))
    s += jnp.sum(p2(base + _BLK, _MID))
    wlast = jnp.where(lcol < lrow, 1.0,
                      jnp.where(lcol == lrow, 0.5, 0.0)).astype(jnp.float32)
    s += jnp.sum(wlast * p2(base + _HALF, _BLK))
    out_ref[0, 0, 0] = s


def kernel(pos, macro_mask, macro_size_x, macro_size_y):
    x = jax.lax.slice(pos, (0,), (_N,))
    y = jax.lax.slice(pos, (_NUM_PHYS,), (_NUM_PHYS + _N,))
    m = macro_mask
    neg = jnp.where(m, jnp.float32(0.0), jnp.float32(-1e7))
    hx = macro_size_x.astype(jnp.float32) * 0.5 + neg
    hy = macro_size_y.astype(jnp.float32) * 0.5 + neg

    col = lambda v: v.reshape(_N, 1)
    dbl = lambda v: jnp.concatenate([v, v]).reshape(1, 2 * _N)

    partial = pl.pallas_call(
        _notch_kernel,
        grid=(_NSTRIP,),
        in_specs=[
            pl.BlockSpec((_BLK, 1), lambda r: (r, 0)),
            pl.BlockSpec((_BLK, 1), lambda r: (r, 0)),
            pl.BlockSpec((_BLK, 1), lambda r: (r, 0)),
            pl.BlockSpec((_BLK, 1), lambda r: (r, 0)),
            pl.BlockSpec((1, 2 * _N), lambda r: (0, 0)),
            pl.BlockSpec((1, 2 * _N), lambda r: (0, 0)),
            pl.BlockSpec((1, 2 * _N), lambda r: (0, 0)),
            pl.BlockSpec((1, 2 * _N), lambda r: (0, 0)),
        ],
        out_shape=jax.ShapeDtypeStruct((_NSTRIP, 1, 1), jnp.float32),
        out_specs=pl.BlockSpec((1, 1, 1), lambda r: (r, 0, 0),
                               memory_space=pltpu.SMEM),
        compiler_params=pltpu.CompilerParams(
            dimension_semantics=("parallel",)),
    )(col(x), col(y), col(hx), col(hy), dbl(x), dbl(y), dbl(hx), dbl(hy))

    total = jnp.sum(partial)
    count = jnp.sum(m.astype(jnp.int32))
    return jnp.where(count < 2, jnp.zeros((), jnp.float32), total)


# speedup vs baseline: 1.2926x; 1.1113x over previous
import jax
import jax.numpy as jnp
from jax.experimental import pallas as pl
from jax.experimental.pallas import tpu as pltpu

_N = 2048
_NUM_PHYS = 600000
_THRESH = 1.0
_BLK = 256
_HALF = _N // 2
_MID = _HALF - _BLK
_NSTRIP = _N // _BLK


def _notch_kernel(gate_ref, xc_ref, yc_ref, hxc_ref, hyc_ref,
                  xr_ref, yr_ref, hxr_ref, hyr_ref, out_ref):
    r = pl.program_id(0)
    base = r * _BLK
    xc = xc_ref[...]
    yc = yc_ref[...]
    hxc = hxc_ref[...]
    hyc = hyc_ref[...]

    def p2(co, w):
        xr = xr_ref[:, pl.ds(co, w)]
        yr = yr_ref[:, pl.ds(co, w)]
        hxr = hxr_ref[:, pl.ds(co, w)]
        hyr = hyr_ref[:, pl.ds(co, w)]
        dx = jnp.maximum(jnp.abs(xc - xr) - (hxc + hxr), 0.0)
        dy = jnp.maximum(jnp.abs(yc - yr) - (hyc + hyr), 0.0)
        p = jnp.maximum((_THRESH - dx) - dy, 0.0)
        return p * p

    lrow = jax.lax.broadcasted_iota(jnp.int32, (_BLK, _BLK), 0)
    lcol = jax.lax.broadcasted_iota(jnp.int32, (_BLK, _BLK), 1)

    s = jnp.sum(jnp.where(lcol > lrow, p2(base, _BLK), 0.0))
    s += jnp.sum(p2(base + _BLK, _MID))
    wlast = jnp.where(lcol < lrow, 1.0,
                      jnp.where(lcol == lrow, 0.5, 0.0)).astype(jnp.float32)
    s += jnp.sum(wlast * p2(base + _HALF, _BLK))

    @pl.when(r == 0)
    def _():
        out_ref[0, 0] = 0.0

    out_ref[0, 0] += s

    @pl.when(r == _NSTRIP - 1)
    def _():
        out_ref[0, 0] = out_ref[0, 0] * gate_ref[0, 0]


def kernel(pos, macro_mask, macro_size_x, macro_size_y):
    x = jax.lax.slice(pos, (0,), (_N,))
    y = jax.lax.slice(pos, (_NUM_PHYS,), (_NUM_PHYS + _N,))
    m = macro_mask
    neg = jnp.where(m, jnp.float32(0.0), jnp.float32(-1e7))
    hx = macro_size_x.astype(jnp.float32) * 0.5 + neg
    hy = macro_size_y.astype(jnp.float32) * 0.5 + neg
    count = jnp.sum(m.astype(jnp.int32))
    gate = jnp.where(count < 2, 0.0, 1.0).astype(jnp.float32).reshape(1, 1)

    col = lambda v: v.reshape(_N, 1)
    dbl = lambda v: jnp.concatenate([v, v]).reshape(1, 2 * _N)

    out = pl.pallas_call(
        _notch_kernel,
        grid=(_NSTRIP,),
        in_specs=[
            pl.BlockSpec(memory_space=pltpu.SMEM),
            pl.BlockSpec((_BLK, 1), lambda r: (r, 0)),
            pl.BlockSpec((_BLK, 1), lambda r: (r, 0)),
            pl.BlockSpec((_BLK, 1), lambda r: (r, 0)),
            pl.BlockSpec((_BLK, 1), lambda r: (r, 0)),
            pl.BlockSpec((1, 2 * _N), lambda r: (0, 0)),
            pl.BlockSpec((1, 2 * _N), lambda r: (0, 0)),
            pl.BlockSpec((1, 2 * _N), lambda r: (0, 0)),
            pl.BlockSpec((1, 2 * _N), lambda r: (0, 0)),
        ],
        out_shape=jax.ShapeDtypeStruct((1, 1), jnp.float32),
        out_specs=pl.BlockSpec(memory_space=pltpu.SMEM),
        compiler_params=pltpu.CompilerParams(
            dimension_semantics=("arbitrary",)),
    )(gate, col(x), col(y), col(hx), col(hy), dbl(x), dbl(y), dbl(hx), dbl(hy))

    return out.reshape(())
